# baseline (speedup 1.0000x reference)
import functools

import jax
import jax.numpy as jnp
from jax import lax
from jax.experimental import pallas as pl
from jax.experimental.pallas import tpu as pltpu
from jax.experimental.pallas import tpu_sc as plsc

B = 4096
L = 200
DIM = 64
MAX_LEN = 100000
NW = 32
BPW = B // NW
IBLK = 64
LANES = 16
GATHER_SPLITS = ((0, 104), (104, 24), (128, 72))

_mesh = plsc.VectorSubcoreMesh(core_axis_name="c", subcore_axis_name="s")


@functools.partial(
    pl.kernel,
    mesh=_mesh,
    compiler_params=pltpu.CompilerParams(use_tc_tiling_on_sc=True),
    out_type=jax.ShapeDtypeStruct((B, L, DIM), jnp.float32),
    scratch_types=[
        pltpu.VMEM((IBLK, L), jnp.int32),
        pltpu.VMEM((L, 2 * DIM), jnp.float32),
        pltpu.VMEM((L, DIM), jnp.float32),
        pltpu.SemaphoreType.DMA,
        pltpu.SemaphoreType.DMA,
    ],
)
def _pe_kernel(x_hbm, idx_hbm, tab_hbm, out_hbm, idx_all, rows_v, xb_v, gsem, xsem):
    wid = lax.axis_index("s") * 2 + lax.axis_index("c")
    b0 = wid * BPW

    def blk_body(blk, carry):
        pltpu.sync_copy(idx_hbm.at[pl.ds(b0 + blk * IBLK, IBLK)], idx_all)

        def chunk_body(cl, carry2):
            c = blk * IBLK + cl
            for off, sz in GATHER_SPLITS:
                pltpu.async_copy(
                    tab_hbm.at[idx_all.at[cl, pl.ds(off, sz)]],
                    rows_v.at[pl.ds(off, sz)],
                    gsem,
                )
            xcp = pltpu.async_copy(x_hbm.at[b0 + c], xb_v, xsem)
            for off, sz in GATHER_SPLITS:
                pltpu.make_async_copy(
                    tab_hbm.at[idx_all.at[cl, pl.ds(off, sz)]],
                    rows_v.at[pl.ds(off, sz)],
                    gsem,
                ).wait()
            xcp.wait()

            def add_body(r, carry3):
                for k in range(DIM // LANES):
                    plsc.addupdate(
                        xb_v.at[r, pl.ds(k * LANES, LANES)],
                        rows_v[r, pl.ds(k * LANES, LANES)],
                    )
                return carry3

            lax.fori_loop(0, L, add_body, 0, unroll=4)
            pltpu.sync_copy(xb_v, out_hbm.at[b0 + c])
            return carry2

        lax.fori_loop(0, IBLK, chunk_body, 0)
        return carry

    lax.fori_loop(0, BPW // IBLK, blk_body, 0)


def kernel(x, order, pos_enc):
    tabp = jnp.concatenate(
        [pos_enc, jnp.zeros((MAX_LEN, DIM), jnp.float32)], axis=1)
    return _pe_kernel(x, order.astype(jnp.int32), tabp)
